# Initial kernel scaffold; baseline (speedup 1.0000x reference)
import jax
import jax.numpy as jnp
from jax.experimental import pallas as pl


def kernel(latent_states, latent_actions):
    raise NotImplementedError("write your pallas kernel here")


# speedup vs baseline: 6.7877x; 6.7877x over previous
import numpy as np
import jax
import jax.numpy as jnp
from jax.experimental import pallas as pl
from jax.experimental.pallas import tpu as pltpu

_STATE_SPACE_SIZE = 10.0
_ACTION_SPACE_SIZE = 5.0
_N_SAMPLES = 2048
_TAIL = 4
_FAR = 64
_PUSH = 4

_BIG = 3.0e38

_key_s, _key_a = jax.random.split(jax.random.key(42))
_STATE_SAMPLES = np.asarray(
    (jax.random.uniform(_key_s, (_N_SAMPLES, 256), dtype=jnp.float32) * 2 - 1)
) * _STATE_SPACE_SIZE
_ACTION_SAMPLES = np.asarray(
    (jax.random.uniform(_key_a, (_N_SAMPLES, 64), dtype=jnp.float32) * 2 - 1)
) * _ACTION_SPACE_SIZE

_BI = 8
_KC = 32


def _dist4_body(k_dim, bt_ref, s_ref, s4_ref):
    n = bt_ref.shape[1]

    part = jnp.zeros((_BI, n), jnp.float32)
    for k in range(k_dim):
        srow = s_ref[:, k:k + 1]
        brow = bt_ref[k:k + 1, :]
        part = part + jnp.abs(srow - brow)

    lane = jax.lax.broadcasted_iota(jnp.int32, (_BI, n), 1)
    d = part
    for t in range(_TAIL):
        m = jnp.min(d, axis=1, keepdims=True)
        cand = jnp.where(d == m, lane, n)
        idx = jnp.min(cand, axis=1, keepdims=True)
        d = jnp.where(lane == idx, _BIG, d)
        s4_ref[:, t:t + 1] = m


def _dist4(samples, latents_t):
    k_dim, n = latents_t.shape
    nb = samples.shape[0] // _BI

    def body(bt_ref, s_ref, s4_ref):
        _dist4_body(k_dim, bt_ref, s_ref, s4_ref)

    return pl.pallas_call(
        body,
        grid=(nb,),
        in_specs=[
            pl.BlockSpec((k_dim, n), lambda i: (0, 0)),
            pl.BlockSpec((_BI, k_dim), lambda i: (i, 0)),
        ],
        out_specs=pl.BlockSpec((_BI, _TAIL), lambda i: (i, 0)),
        out_shape=jax.ShapeDtypeStruct((samples.shape[0], _TAIL), jnp.float32),
    )(latents_t, samples)


def _finalize_body(ls_ref, la_ref, s_tail_ref, s_sq_ref, a_tail_ref, a_sq_ref,
                   out_ref):
    s_norm = jnp.sum(jnp.abs(ls_ref[:, :]), axis=1)
    a_norm = jnp.sum(jnp.abs(la_ref[:, :]), axis=1)
    s_viol = jnp.maximum(s_norm - _STATE_SPACE_SIZE, 0.0)
    a_viol = jnp.maximum(a_norm - _ACTION_SPACE_SIZE, 0.0)
    size_loss = jnp.mean(s_viol * s_viol) + jnp.mean(a_viol * a_viol)

    rows, cols = s_tail_ref.shape
    fidx = jax.lax.broadcasted_iota(jnp.int32, (rows, cols), 0) * cols + \
        jax.lax.broadcasted_iota(jnp.int32, (rows, cols), 1)
    nfl = rows * cols

    def top64_sum(tail, sq):
        def step(_, carry):
            t, acc = carry
            m = jnp.max(t)
            cand = jnp.where(t == m, fidx, nfl)
            cidx = jnp.min(cand)
            hit = fidx == cidx
            acc = acc + jnp.sum(jnp.where(hit, sq, 0.0))
            t = jnp.where(hit, -_BIG, t)
            return t, acc
        _, acc = jax.lax.fori_loop(0, _FAR, step, (tail, jnp.float32(0.0)))
        return acc

    s_cov = top64_sum(s_tail_ref[:, :], s_sq_ref[:, :]) / (_FAR * _PUSH)
    a_cov = top64_sum(a_tail_ref[:, :], a_sq_ref[:, :]) / (_FAR * _PUSH)

    out_ref[:, :] = jnp.broadcast_to(size_loss + s_cov + a_cov, (1, 1))


def _finalize(ls, la, s_tail, s_sq, a_tail, a_sq):
    return pl.pallas_call(
        _finalize_body,
        out_shape=jax.ShapeDtypeStruct((1, 1), jnp.float32),
    )(ls, la, s_tail, s_sq, a_tail, a_sq)


@jax.jit
def kernel(latent_states, latent_actions):
    ls = latent_states.reshape(-1, latent_states.shape[-1])
    la = latent_actions.reshape(-1, latent_actions.shape[-1])

    s4_state = _dist4(jnp.asarray(_STATE_SAMPLES), ls.T)
    s4_action = _dist4(jnp.asarray(_ACTION_SAMPLES), la.T)

    s_tail = s4_state.mean(axis=1).reshape(16, 128)
    s_sq = (s4_state * s4_state).sum(axis=1).reshape(16, 128)
    a_tail = s4_action.mean(axis=1).reshape(16, 128)
    a_sq = (s4_action * s4_action).sum(axis=1).reshape(16, 128)

    out = _finalize(ls, la, s_tail, s_sq, a_tail, a_sq)
    return out[0, 0]


# speedup vs baseline: 8.7617x; 1.2908x over previous
import numpy as np
import jax
import jax.numpy as jnp
from jax.experimental import pallas as pl
from jax.experimental.pallas import tpu as pltpu

_STATE_SPACE_SIZE = 10.0
_ACTION_SPACE_SIZE = 5.0
_N_SAMPLES = 2048
_TAIL = 4
_FAR = 64
_PUSH = 4

_BIG = 3.0e38

def _space_samples(state_dim, action_dim):
    key_s, key_a = jax.random.split(jax.random.key(42))
    s = (jax.random.uniform(key_s, (_N_SAMPLES, state_dim),
                            dtype=jnp.float32) * 2 - 1) * _STATE_SPACE_SIZE
    a = (jax.random.uniform(key_a, (_N_SAMPLES, action_dim),
                            dtype=jnp.float32) * 2 - 1) * _ACTION_SPACE_SIZE
    return s, a

_BI = 8
_KC = 32


def _dist4_body(k_dim, bt_ref, s_ref, s4_ref):
    n = bt_ref.shape[1]

    part0 = jnp.zeros((_BI, n), jnp.float32)
    part1 = jnp.zeros((_BI, n), jnp.float32)
    for k in range(0, k_dim, 2):
        srow0 = s_ref[:, k:k + 1]
        brow0 = bt_ref[k * _BI:k * _BI + _BI, :]
        part0 = part0 + jnp.abs(srow0 - brow0)
        srow1 = s_ref[:, k + 1:k + 2]
        brow1 = bt_ref[(k + 1) * _BI:(k + 2) * _BI, :]
        part1 = part1 + jnp.abs(srow1 - brow1)

    lane = jax.lax.broadcasted_iota(jnp.int32, (_BI, n), 1)
    d = part0 + part1
    for t in range(_TAIL):
        m = jnp.min(d, axis=1, keepdims=True)
        cand = jnp.where(d == m, lane, n)
        idx = jnp.min(cand, axis=1, keepdims=True)
        d = jnp.where(lane == idx, _BIG, d)
        s4_ref[:, t:t + 1] = m


def _dist4(samples, latents_t):
    k_dim, n = latents_t.shape
    nb = samples.shape[0] // _BI

    bt8 = jnp.broadcast_to(latents_t[:, None, :], (k_dim, _BI, n))
    bt8 = bt8.reshape(k_dim * _BI, n)

    def body(bt_ref, s_ref, s4_ref):
        _dist4_body(k_dim, bt_ref, s_ref, s4_ref)

    return pl.pallas_call(
        body,
        grid=(nb,),
        in_specs=[
            pl.BlockSpec((k_dim * _BI, n), lambda i: (0, 0)),
            pl.BlockSpec((_BI, k_dim), lambda i: (i, 0)),
        ],
        out_specs=pl.BlockSpec((_BI, _TAIL), lambda i: (i, 0)),
        out_shape=jax.ShapeDtypeStruct((samples.shape[0], _TAIL), jnp.float32),
    )(bt8, samples)


def _finalize_body(ls_ref, la_ref, s_tail_ref, s_sq_ref, a_tail_ref, a_sq_ref,
                   out_ref):
    s_norm = jnp.sum(jnp.abs(ls_ref[:, :]), axis=1)
    a_norm = jnp.sum(jnp.abs(la_ref[:, :]), axis=1)
    s_viol = jnp.maximum(s_norm - _STATE_SPACE_SIZE, 0.0)
    a_viol = jnp.maximum(a_norm - _ACTION_SPACE_SIZE, 0.0)
    size_loss = jnp.mean(s_viol * s_viol) + jnp.mean(a_viol * a_viol)

    rows, cols = s_tail_ref.shape
    fidx = jax.lax.broadcasted_iota(jnp.int32, (rows, cols), 0) * cols + \
        jax.lax.broadcasted_iota(jnp.int32, (rows, cols), 1)
    nfl = rows * cols

    def top64_sum(tail, sq):
        def step(_, carry):
            t, acc = carry
            m = jnp.max(t)
            cand = jnp.where(t == m, fidx, nfl)
            cidx = jnp.min(cand)
            hit = fidx == cidx
            acc = acc + jnp.sum(jnp.where(hit, sq, 0.0))
            t = jnp.where(hit, -_BIG, t)
            return t, acc
        _, acc = jax.lax.fori_loop(0, _FAR, step, (tail, jnp.float32(0.0)))
        return acc

    s_cov = top64_sum(s_tail_ref[:, :], s_sq_ref[:, :]) / (_FAR * _PUSH)
    a_cov = top64_sum(a_tail_ref[:, :], a_sq_ref[:, :]) / (_FAR * _PUSH)

    out_ref[:, :] = jnp.broadcast_to(size_loss + s_cov + a_cov, (1, 1))


def _finalize(ls, la, s_tail, s_sq, a_tail, a_sq):
    return pl.pallas_call(
        _finalize_body,
        out_shape=jax.ShapeDtypeStruct((1, 1), jnp.float32),
    )(ls, la, s_tail, s_sq, a_tail, a_sq)


@jax.jit
def kernel(latent_states, latent_actions):
    ls = latent_states.reshape(-1, latent_states.shape[-1])
    la = latent_actions.reshape(-1, latent_actions.shape[-1])

    s_samples, a_samples = _space_samples(ls.shape[-1], la.shape[-1])
    s4_state = _dist4(s_samples, ls.T)
    s4_action = _dist4(a_samples, la.T)

    s_tail = s4_state.mean(axis=1).reshape(16, 128)
    s_sq = (s4_state * s4_state).sum(axis=1).reshape(16, 128)
    a_tail = s4_action.mean(axis=1).reshape(16, 128)
    a_sq = (s4_action * s4_action).sum(axis=1).reshape(16, 128)

    out = _finalize(ls, la, s_tail, s_sq, a_tail, a_sq)
    return out[0, 0]


# speedup vs baseline: 10.4395x; 1.1915x over previous
import numpy as np
import jax
import jax.numpy as jnp
from jax.experimental import pallas as pl
from jax.experimental.pallas import tpu as pltpu

_STATE_SPACE_SIZE = 10.0
_ACTION_SPACE_SIZE = 5.0
_N_SAMPLES = 2048
_TAIL = 4
_FAR = 64
_PUSH = 4

_BIG = 3.0e38

def _space_samples(state_dim, action_dim):
    key_s, key_a = jax.random.split(jax.random.key(42))
    s = (jax.random.uniform(key_s, (_N_SAMPLES, state_dim),
                            dtype=jnp.float32) * 2 - 1) * _STATE_SPACE_SIZE
    a = (jax.random.uniform(key_a, (_N_SAMPLES, action_dim),
                            dtype=jnp.float32) * 2 - 1) * _ACTION_SPACE_SIZE
    return s, a

_BI = 8
_KC = 32


def _dist_accum(k_dim, bt_ref, s_ref):
    n = bt_ref.shape[1]
    part0 = jnp.zeros((_BI, n), jnp.float32)
    part1 = jnp.zeros((_BI, n), jnp.float32)
    for k in range(0, k_dim, 2):
        srow0 = s_ref[:, k:k + 1]
        brow0 = bt_ref[k * _BI:k * _BI + _BI, :]
        part0 = part0 + jnp.abs(srow0 - brow0)
        srow1 = s_ref[:, k + 1:k + 2]
        brow1 = bt_ref[(k + 1) * _BI:(k + 2) * _BI, :]
        part1 = part1 + jnp.abs(srow1 - brow1)
    return part0 + part1


def _min4_extract(d, s4_ref):
    n = d.shape[1]
    lane = jax.lax.broadcasted_iota(jnp.int32, (_BI, n), 1)
    for t in range(_TAIL):
        m = jnp.min(d, axis=1, keepdims=True)
        cand = jnp.where(d == m, lane, n)
        idx = jnp.min(cand, axis=1, keepdims=True)
        d = jnp.where(lane == idx, _BIG, d)
        s4_ref[:, t:t + 1] = m


def _dist4_pair(s_samples, a_samples, ls_t, la_t):
    ks, n = ls_t.shape
    ka = la_t.shape[0]
    nb = _N_SAMPLES // _BI

    def bcast8(bt, k_dim):
        b = jnp.broadcast_to(bt[:, None, :], (k_dim, _BI, n))
        return b.reshape(k_dim * _BI, n)

    def body(bts_ref, ss_ref, bta_ref, sa_ref, s4s_ref, s4a_ref):
        ds = _dist_accum(ks, bts_ref, ss_ref)
        da = _dist_accum(ka, bta_ref, sa_ref)
        _min4_extract(ds, s4s_ref)
        _min4_extract(da, s4a_ref)

    out = jax.ShapeDtypeStruct((_N_SAMPLES, _TAIL), jnp.float32)
    return pl.pallas_call(
        body,
        grid=(nb,),
        in_specs=[
            pl.BlockSpec((ks * _BI, n), lambda i: (0, 0)),
            pl.BlockSpec((_BI, ks), lambda i: (i, 0)),
            pl.BlockSpec((ka * _BI, n), lambda i: (0, 0)),
            pl.BlockSpec((_BI, ka), lambda i: (i, 0)),
        ],
        out_specs=[
            pl.BlockSpec((_BI, _TAIL), lambda i: (i, 0)),
            pl.BlockSpec((_BI, _TAIL), lambda i: (i, 0)),
        ],
        out_shape=[out, out],
        compiler_params=pltpu.CompilerParams(
            dimension_semantics=("arbitrary",),
        ),
    )(bcast8(ls_t, ks), s_samples, bcast8(la_t, ka), a_samples)


def _finalize_body(ls_ref, la_ref, s_tail_ref, s_sq_ref, a_tail_ref, a_sq_ref,
                   out_ref):
    s_norm = jnp.sum(jnp.abs(ls_ref[:, :]), axis=1)
    a_norm = jnp.sum(jnp.abs(la_ref[:, :]), axis=1)
    s_viol = jnp.maximum(s_norm - _STATE_SPACE_SIZE, 0.0)
    a_viol = jnp.maximum(a_norm - _ACTION_SPACE_SIZE, 0.0)
    size_loss = jnp.mean(s_viol * s_viol) + jnp.mean(a_viol * a_viol)

    rows, cols = s_tail_ref.shape
    fidx = jax.lax.broadcasted_iota(jnp.int32, (rows, cols), 0) * cols + \
        jax.lax.broadcasted_iota(jnp.int32, (rows, cols), 1)
    nfl = rows * cols

    def top64_sum(tail, sq):
        def step(_, carry):
            t, acc = carry
            m = jnp.max(t)
            cand = jnp.where(t == m, fidx, nfl)
            cidx = jnp.min(cand)
            hit = fidx == cidx
            acc = acc + jnp.sum(jnp.where(hit, sq, 0.0))
            t = jnp.where(hit, -_BIG, t)
            return t, acc
        _, acc = jax.lax.fori_loop(0, _FAR, step, (tail, jnp.float32(0.0)))
        return acc

    s_cov = top64_sum(s_tail_ref[:, :], s_sq_ref[:, :]) / (_FAR * _PUSH)
    a_cov = top64_sum(a_tail_ref[:, :], a_sq_ref[:, :]) / (_FAR * _PUSH)

    out_ref[:, :] = jnp.broadcast_to(size_loss + s_cov + a_cov, (1, 1))


def _finalize(ls, la, s_tail, s_sq, a_tail, a_sq):
    return pl.pallas_call(
        _finalize_body,
        out_shape=jax.ShapeDtypeStruct((1, 1), jnp.float32),
    )(ls, la, s_tail, s_sq, a_tail, a_sq)


@jax.jit
def kernel(latent_states, latent_actions):
    ls = latent_states.reshape(-1, latent_states.shape[-1])
    la = latent_actions.reshape(-1, latent_actions.shape[-1])

    s_samples, a_samples = _space_samples(ls.shape[-1], la.shape[-1])
    s4_state, s4_action = _dist4_pair(s_samples, a_samples, ls.T, la.T)

    s_tail = s4_state.mean(axis=1).reshape(16, 128)
    s_sq = (s4_state * s4_state).sum(axis=1).reshape(16, 128)
    a_tail = s4_action.mean(axis=1).reshape(16, 128)
    a_sq = (s4_action * s4_action).sum(axis=1).reshape(16, 128)

    out = _finalize(ls, la, s_tail, s_sq, a_tail, a_sq)
    return out[0, 0]
